# baseline (speedup 1.0000x reference)
import jax
import jax.numpy as jnp
from jax import lax
from jax.experimental import pallas as pl
from jax.experimental.pallas import tpu as pltpu
from jax.experimental.pallas import tpu_sc as plsc

LENGTH = 1_000_000
B, H = 16384, 200
NC, NS = 2, 16
NW = NC * NS
ROWS_PER_W = B // NW
WR = 8
WINDOWS = ROWS_PER_W // WR
SEG = 62_496
PIECES = 4
PIECE = SEG // PIECES
TAIL = LENGTH - NS * SEG


def _body(idx_hbm, u_hbm, out_hbm, u_sp, bounce_v, idx_v, out_v,
          sem_g0, sem_g1, sem_i0, sem_i1, sem_o0, sem_o1, sem_si, sem_so):
    c = lax.axis_index("c")
    s = lax.axis_index("s")
    wid = s * NC + c
    row0 = wid * ROWS_PER_W
    sem_g = (sem_g0, sem_g1)
    sem_i = (sem_i0, sem_i1)
    sem_o = (sem_o0, sem_o1)

    def idx_rows(wi):
        return idx_hbm.at[pl.ds(row0 + wi * WR, WR)]

    def out_rows(wi):
        return out_hbm.at[pl.ds(row0 + wi * WR, WR)]

    def fire_gathers(wi, b):
        for j in range(WR):
            irow = idx_v.at[b].at[j]
            orow = out_v.at[b].at[j]
            pltpu.async_copy(u_sp.at[irow.at[pl.ds(0, 128)]],
                             orow.at[pl.ds(0, 128)], sem_g[b])
            pltpu.async_copy(u_sp.at[irow.at[pl.ds(128, H - 128)]],
                             orow.at[pl.ds(128, H - 128)], sem_g[b])

    def drain_gathers(b):
        for j in range(WR):
            pltpu.make_async_copy(u_sp.at[idx_v.at[b].at[j].at[pl.ds(0, 128)]],
                                  out_v.at[b].at[j].at[pl.ds(0, 128)],
                                  sem_g[b]).wait()
            pltpu.make_async_copy(
                u_sp.at[idx_v.at[b].at[j].at[pl.ds(128, H - 128)]],
                out_v.at[b].at[j].at[pl.ds(128, H - 128)], sem_g[b]).wait()

    pltpu.async_copy(idx_rows(0), idx_v.at[0], sem_i0)
    pltpu.async_copy(idx_rows(1), idx_v.at[1], sem_i1)

    seg0 = s * SEG

    @pl.when(s == NS - 1)
    def _():
        pltpu.sync_copy(u_hbm.at[pl.ds(NS * SEG, TAIL)],
                        bounce_v.at[pl.ds(0, TAIL)])
        pltpu.sync_copy(bounce_v.at[pl.ds(0, TAIL)],
                        u_sp.at[pl.ds(NS * SEG, TAIL)])

    def piece_in(p):
        return pltpu.make_async_copy(
            u_hbm.at[pl.ds(seg0 + p * PIECE, PIECE)],
            bounce_v.at[pl.ds((p % 2) * PIECE, PIECE)], sem_si)

    def piece_out(p):
        return pltpu.make_async_copy(
            bounce_v.at[pl.ds((p % 2) * PIECE, PIECE)],
            u_sp.at[pl.ds(seg0 + p * PIECE, PIECE)], sem_so)

    piece_in(0).start()
    for p in range(PIECES):
        piece_in(p).wait()
        if p + 1 < PIECES:
            if p >= 1:
                piece_out(p - 1).wait()
            piece_in(p + 1).start()
        piece_out(p).start()
    piece_out(PIECES - 2).wait()
    piece_out(PIECES - 1).wait()

    plsc.subcore_barrier()

    @pl.loop(0, WINDOWS, step=2)
    def _(w):
        for b in range(2):
            wi = w + b
            pltpu.make_async_copy(idx_rows(wi), idx_v.at[b], sem_i[b]).wait()
            @pl.when(wi >= 2)
            def _():
                pltpu.make_async_copy(out_v.at[b], out_rows(wi - 2),
                                      sem_o[b]).wait()
            fire_gathers(wi, b)
            @pl.when(wi >= 1)
            def _():
                drain_gathers(1 - b)
                @pl.when(wi + 1 < WINDOWS)
                def _():
                    pltpu.async_copy(idx_rows(wi + 1), idx_v.at[1 - b],
                                     sem_i[1 - b])
                pltpu.async_copy(out_v.at[1 - b], out_rows(wi - 1),
                                sem_o[1 - b])

    drain_gathers(1)
    pltpu.async_copy(out_v.at[1], out_rows(WINDOWS - 1), sem_o1)
    pltpu.make_async_copy(out_v.at[0], out_rows(WINDOWS - 2), sem_o0).wait()
    pltpu.make_async_copy(out_v.at[1], out_rows(WINDOWS - 1), sem_o1).wait()


def kernel(idx, u):
    return pl.kernel(
        _body,
        out_type=jax.ShapeDtypeStruct((B, H), jnp.float32),
        mesh=plsc.VectorSubcoreMesh(core_axis_name="c", subcore_axis_name="s"),
        scratch_types=[
            pltpu.VMEM_SHARED((LENGTH,), jnp.float32),
            pltpu.VMEM((2 * PIECE,), jnp.float32),
            pltpu.VMEM((2, WR, H), jnp.int32),
            pltpu.VMEM((2, WR, H), jnp.float32),
            pltpu.SemaphoreType.DMA,
            pltpu.SemaphoreType.DMA,
            pltpu.SemaphoreType.DMA,
            pltpu.SemaphoreType.DMA,
            pltpu.SemaphoreType.DMA,
            pltpu.SemaphoreType.DMA,
            pltpu.SemaphoreType.DMA,
            pltpu.SemaphoreType.DMA,
        ],
    )(idx, u)


# speedup vs baseline: 1.0845x; 1.0845x over previous
import jax
import jax.numpy as jnp
from jax import lax
from jax.experimental import pallas as pl
from jax.experimental.pallas import tpu as pltpu
from jax.experimental.pallas import tpu_sc as plsc

LENGTH = 1_000_000
B, H = 16384, 200
NC, NS = 2, 16
NW = NC * NS
ROWS_PER_W = B // NW
WR = 8
WINDOWS = ROWS_PER_W // WR
SEG = 62_496
PIECES = 4
PIECE = SEG // PIECES
TAIL = LENGTH - NS * SEG


def _body(idx_hbm, u_hbm, out_hbm, u_sp, bounce_v, idx_v, out_v,
          sem_g, sem_i0, sem_i1, sem_o0, sem_o1, sem_si, sem_so):
    c = lax.axis_index("c")
    s = lax.axis_index("s")
    wid = s * NC + c
    row0 = wid * ROWS_PER_W
    sem_i = (sem_i0, sem_i1)
    sem_o = (sem_o0, sem_o1)

    def idx_rows(wi):
        return idx_hbm.at[pl.ds(row0 + wi * WR, WR)]

    def out_rows(wi):
        return out_hbm.at[pl.ds(row0 + wi * WR, WR)]

    pltpu.async_copy(idx_rows(0), idx_v.at[0], sem_i0)
    pltpu.async_copy(idx_rows(1), idx_v.at[1], sem_i1)

    seg0 = s * SEG

    @pl.when(s == NS - 1)
    def _():
        pltpu.sync_copy(u_hbm.at[pl.ds(NS * SEG, TAIL)],
                        bounce_v.at[pl.ds(0, TAIL)])
        pltpu.sync_copy(bounce_v.at[pl.ds(0, TAIL)],
                        u_sp.at[pl.ds(NS * SEG, TAIL)])

    def piece_in(p):
        return pltpu.make_async_copy(
            u_hbm.at[pl.ds(seg0 + p * PIECE, PIECE)],
            bounce_v.at[pl.ds((p % 2) * PIECE, PIECE)], sem_si)

    def piece_out(p):
        return pltpu.make_async_copy(
            bounce_v.at[pl.ds((p % 2) * PIECE, PIECE)],
            u_sp.at[pl.ds(seg0 + p * PIECE, PIECE)], sem_so)

    piece_in(0).start()
    for p in range(PIECES):
        piece_in(p).wait()
        if p + 1 < PIECES:
            if p >= 1:
                piece_out(p - 1).wait()
            piece_in(p + 1).start()
        piece_out(p).start()
    piece_out(PIECES - 2).wait()
    piece_out(PIECES - 1).wait()

    plsc.subcore_barrier()

    @pl.loop(0, WINDOWS, step=2)
    def _(w):
        for b in range(2):
            wi = w + b
            pltpu.make_async_copy(idx_rows(wi), idx_v.at[b], sem_i[b]).wait()
            @pl.when(wi >= 2)
            def _():
                pltpu.make_async_copy(out_v.at[b], out_rows(wi - 2),
                                      sem_o[b]).wait()
            cps = []
            for j in range(WR):
                irow = idx_v.at[b].at[j]
                orow = out_v.at[b].at[j]
                cps.append(pltpu.async_copy(
                    u_sp.at[irow.at[pl.ds(0, 128)]],
                    orow.at[pl.ds(0, 128)], sem_g))
                cps.append(pltpu.async_copy(
                    u_sp.at[irow.at[pl.ds(128, H - 128)]],
                    orow.at[pl.ds(128, H - 128)], sem_g))
            for cp in cps:
                cp.wait()
            @pl.when(wi + 2 < WINDOWS)
            def _():
                pltpu.async_copy(idx_rows(wi + 2), idx_v.at[b], sem_i[b])
            pltpu.async_copy(out_v.at[b], out_rows(wi), sem_o[b])

    pltpu.make_async_copy(out_v.at[0], out_rows(WINDOWS - 2), sem_o0).wait()
    pltpu.make_async_copy(out_v.at[1], out_rows(WINDOWS - 1), sem_o1).wait()


def kernel(idx, u):
    return pl.kernel(
        _body,
        out_type=jax.ShapeDtypeStruct((B, H), jnp.float32),
        mesh=plsc.VectorSubcoreMesh(core_axis_name="c", subcore_axis_name="s"),
        scratch_types=[
            pltpu.VMEM_SHARED((LENGTH,), jnp.float32),
            pltpu.VMEM((2 * PIECE,), jnp.float32),
            pltpu.VMEM((2, WR, H), jnp.int32),
            pltpu.VMEM((2, WR, H), jnp.float32),
            pltpu.SemaphoreType.DMA,
            pltpu.SemaphoreType.DMA,
            pltpu.SemaphoreType.DMA,
            pltpu.SemaphoreType.DMA,
            pltpu.SemaphoreType.DMA,
            pltpu.SemaphoreType.DMA,
            pltpu.SemaphoreType.DMA,
        ],
    )(idx, u)


# speedup vs baseline: 1.1950x; 1.1019x over previous
import jax
import jax.numpy as jnp
from jax import lax
from jax.experimental import pallas as pl
from jax.experimental.pallas import tpu as pltpu
from jax.experimental.pallas import tpu_sc as plsc

LENGTH = 1_000_000
B, H = 16384, 200
NC, NS = 2, 16
NW = NC * NS
ROWS_PER_W = B // NW
WR = 16
WINDOWS = ROWS_PER_W // WR
SEG = 62_496
PIECES = 4
PIECE = SEG // PIECES
TAIL = LENGTH - NS * SEG


def _body(idx_hbm, u_hbm, out_hbm, u_sp, bounce_v, idx_v, out_v,
          sem_g, sem_i0, sem_i1, sem_o0, sem_o1, sem_si, sem_so):
    c = lax.axis_index("c")
    s = lax.axis_index("s")
    wid = s * NC + c
    row0 = wid * ROWS_PER_W
    sem_i = (sem_i0, sem_i1)
    sem_o = (sem_o0, sem_o1)

    def idx_rows(wi):
        return idx_hbm.at[pl.ds(row0 + wi * WR, WR)]

    def out_rows(wi):
        return out_hbm.at[pl.ds(row0 + wi * WR, WR)]

    pltpu.async_copy(idx_rows(0), idx_v.at[0], sem_i0)
    pltpu.async_copy(idx_rows(1), idx_v.at[1], sem_i1)

    seg0 = s * SEG

    @pl.when(s == NS - 1)
    def _():
        pltpu.sync_copy(u_hbm.at[pl.ds(NS * SEG, TAIL)],
                        bounce_v.at[pl.ds(0, TAIL)])
        pltpu.sync_copy(bounce_v.at[pl.ds(0, TAIL)],
                        u_sp.at[pl.ds(NS * SEG, TAIL)])

    def piece_in(p):
        return pltpu.make_async_copy(
            u_hbm.at[pl.ds(seg0 + p * PIECE, PIECE)],
            bounce_v.at[pl.ds((p % 2) * PIECE, PIECE)], sem_si)

    def piece_out(p):
        return pltpu.make_async_copy(
            bounce_v.at[pl.ds((p % 2) * PIECE, PIECE)],
            u_sp.at[pl.ds(seg0 + p * PIECE, PIECE)], sem_so)

    piece_in(0).start()
    for p in range(PIECES):
        piece_in(p).wait()
        if p + 1 < PIECES:
            if p >= 1:
                piece_out(p - 1).wait()
            piece_in(p + 1).start()
        piece_out(p).start()
    piece_out(PIECES - 2).wait()
    piece_out(PIECES - 1).wait()

    plsc.subcore_barrier()

    @pl.loop(0, WINDOWS, step=2)
    def _(w):
        for b in range(2):
            wi = w + b
            pltpu.make_async_copy(idx_rows(wi), idx_v.at[b], sem_i[b]).wait()
            @pl.when(wi >= 2)
            def _():
                pltpu.make_async_copy(out_v.at[b], out_rows(wi - 2),
                                      sem_o[b]).wait()
            cps = []
            for j in range(WR):
                irow = idx_v.at[b].at[j]
                orow = out_v.at[b].at[j]
                cps.append(pltpu.async_copy(
                    u_sp.at[irow.at[pl.ds(0, 128)]],
                    orow.at[pl.ds(0, 128)], sem_g))
                cps.append(pltpu.async_copy(
                    u_sp.at[irow.at[pl.ds(128, H - 128)]],
                    orow.at[pl.ds(128, H - 128)], sem_g))
            for cp in cps:
                cp.wait()
            @pl.when(wi + 2 < WINDOWS)
            def _():
                pltpu.async_copy(idx_rows(wi + 2), idx_v.at[b], sem_i[b])
            pltpu.async_copy(out_v.at[b], out_rows(wi), sem_o[b])

    pltpu.make_async_copy(out_v.at[0], out_rows(WINDOWS - 2), sem_o0).wait()
    pltpu.make_async_copy(out_v.at[1], out_rows(WINDOWS - 1), sem_o1).wait()


def kernel(idx, u):
    return pl.kernel(
        _body,
        out_type=jax.ShapeDtypeStruct((B, H), jnp.float32),
        mesh=plsc.VectorSubcoreMesh(core_axis_name="c", subcore_axis_name="s"),
        scratch_types=[
            pltpu.VMEM_SHARED((LENGTH,), jnp.float32),
            pltpu.VMEM((2 * PIECE,), jnp.float32),
            pltpu.VMEM((2, WR, H), jnp.int32),
            pltpu.VMEM((2, WR, H), jnp.float32),
            pltpu.SemaphoreType.DMA,
            pltpu.SemaphoreType.DMA,
            pltpu.SemaphoreType.DMA,
            pltpu.SemaphoreType.DMA,
            pltpu.SemaphoreType.DMA,
            pltpu.SemaphoreType.DMA,
            pltpu.SemaphoreType.DMA,
        ],
    )(idx, u)


# speedup vs baseline: 1.2100x; 1.0126x over previous
import jax
import jax.numpy as jnp
from jax import lax
from jax.experimental import pallas as pl
from jax.experimental.pallas import tpu as pltpu
from jax.experimental.pallas import tpu_sc as plsc

LENGTH = 1_000_000
B, H = 16384, 200
NC, NS = 2, 16
NW = NC * NS
ROWS_PER_W = B // NW
WR = 32
WINDOWS = ROWS_PER_W // WR
SEG = 62_496
PIECES = 4
PIECE = SEG // PIECES
TAIL = LENGTH - NS * SEG


def _body(idx_hbm, u_hbm, out_hbm, u_sp, bounce_v, idx_v, out_v,
          sem_g, sem_i0, sem_i1, sem_o0, sem_o1, sem_si, sem_so):
    c = lax.axis_index("c")
    s = lax.axis_index("s")
    wid = s * NC + c
    row0 = wid * ROWS_PER_W
    sem_i = (sem_i0, sem_i1)
    sem_o = (sem_o0, sem_o1)

    def idx_rows(wi):
        return idx_hbm.at[pl.ds(row0 + wi * WR, WR)]

    def out_rows(wi):
        return out_hbm.at[pl.ds(row0 + wi * WR, WR)]

    pltpu.async_copy(idx_rows(0), idx_v.at[0], sem_i0)
    pltpu.async_copy(idx_rows(1), idx_v.at[1], sem_i1)

    seg0 = s * SEG

    @pl.when(s == NS - 1)
    def _():
        pltpu.sync_copy(u_hbm.at[pl.ds(NS * SEG, TAIL)],
                        bounce_v.at[pl.ds(0, TAIL)])
        pltpu.sync_copy(bounce_v.at[pl.ds(0, TAIL)],
                        u_sp.at[pl.ds(NS * SEG, TAIL)])

    def piece_in(p):
        return pltpu.make_async_copy(
            u_hbm.at[pl.ds(seg0 + p * PIECE, PIECE)],
            bounce_v.at[pl.ds((p % 2) * PIECE, PIECE)], sem_si)

    def piece_out(p):
        return pltpu.make_async_copy(
            bounce_v.at[pl.ds((p % 2) * PIECE, PIECE)],
            u_sp.at[pl.ds(seg0 + p * PIECE, PIECE)], sem_so)

    piece_in(0).start()
    for p in range(PIECES):
        piece_in(p).wait()
        if p + 1 < PIECES:
            if p >= 1:
                piece_out(p - 1).wait()
            piece_in(p + 1).start()
        piece_out(p).start()
    piece_out(PIECES - 2).wait()
    piece_out(PIECES - 1).wait()

    plsc.subcore_barrier()

    @pl.loop(0, WINDOWS, step=2)
    def _(w):
        for b in range(2):
            wi = w + b
            pltpu.make_async_copy(idx_rows(wi), idx_v.at[b], sem_i[b]).wait()
            @pl.when(wi >= 2)
            def _():
                pltpu.make_async_copy(out_v.at[b], out_rows(wi - 2),
                                      sem_o[b]).wait()
            cps = []
            for j in range(WR):
                irow = idx_v.at[b].at[j]
                orow = out_v.at[b].at[j]
                cps.append(pltpu.async_copy(
                    u_sp.at[irow.at[pl.ds(0, 128)]],
                    orow.at[pl.ds(0, 128)], sem_g))
                cps.append(pltpu.async_copy(
                    u_sp.at[irow.at[pl.ds(128, H - 128)]],
                    orow.at[pl.ds(128, H - 128)], sem_g))
            for cp in cps:
                cp.wait()
            @pl.when(wi + 2 < WINDOWS)
            def _():
                pltpu.async_copy(idx_rows(wi + 2), idx_v.at[b], sem_i[b])
            pltpu.async_copy(out_v.at[b], out_rows(wi), sem_o[b])

    pltpu.make_async_copy(out_v.at[0], out_rows(WINDOWS - 2), sem_o0).wait()
    pltpu.make_async_copy(out_v.at[1], out_rows(WINDOWS - 1), sem_o1).wait()


def kernel(idx, u):
    return pl.kernel(
        _body,
        out_type=jax.ShapeDtypeStruct((B, H), jnp.float32),
        mesh=plsc.VectorSubcoreMesh(core_axis_name="c", subcore_axis_name="s"),
        scratch_types=[
            pltpu.VMEM_SHARED((LENGTH,), jnp.float32),
            pltpu.VMEM((2 * PIECE,), jnp.float32),
            pltpu.VMEM((2, WR, H), jnp.int32),
            pltpu.VMEM((2, WR, H), jnp.float32),
            pltpu.SemaphoreType.DMA,
            pltpu.SemaphoreType.DMA,
            pltpu.SemaphoreType.DMA,
            pltpu.SemaphoreType.DMA,
            pltpu.SemaphoreType.DMA,
            pltpu.SemaphoreType.DMA,
            pltpu.SemaphoreType.DMA,
        ],
    )(idx, u)
